# baseline (speedup 1.0000x reference)
import functools

import jax
import jax.numpy as jnp
from jax import lax
from jax.experimental import pallas as pl
from jax.experimental.pallas import tpu as pltpu
from jax.experimental.pallas import tpu_sc as plsc

B = 16384
HIST = 50
D = 16
NC = 2
NS = 16
NW = NC * NS
BW = B // NW
G = 16
CH = G * HIST
S = 80
NOPS = CH // S
NG = BW // G
UC = 128
NU = BW // UC

_mesh = plsc.VectorSubcoreMesh(
    core_axis_name="c", subcore_axis_name="s", num_cores=NC, num_subcores=NS
)

_f32 = jnp.float32
_out_sds = jax.ShapeDtypeStruct((B, D), _f32)


@functools.partial(
    pl.kernel,
    out_type=(_out_sds, _out_sds, _out_sds, _out_sds),
    mesh=_mesh,
    compiler_params=pltpu.CompilerParams(use_tc_tiling_on_sc=False),
    scratch_types=[
        pltpu.VMEM((BW,), jnp.int32),
        pltpu.VMEM((BW,), jnp.int32),
        pltpu.VMEM((BW,), jnp.int32),
        pltpu.VMEM((BW * HIST,), jnp.int32),
        pltpu.VMEM((BW, D), _f32),
        pltpu.VMEM((BW, D), _f32),
        pltpu.VMEM((BW, D), _f32),
        pltpu.VMEM((BW, D), _f32),
        pltpu.VMEM((CH, D), _f32),
        pltpu.VMEM((CH, D), _f32),
        pltpu.SemaphoreType.DMA,
        pltpu.SemaphoreType.DMA,
        pltpu.SemaphoreType.DMA,
        pltpu.SemaphoreType.DMA,
    ],
)
def _emb_kernel(u_hbm, i_hbm, c_hbm, hf_hbm, wu_hbm, wi_hbm, wc_hbm, wh_hbm,
                ou_hbm, oi_hbm, oc_hbm, oh_hbm,
                uidx, iidx, cidx, hidx, urow, irow, crow, hout, hb0, hb1,
                s_in, s_uic, s0, s1):
    cid = lax.axis_index("c")
    sid = lax.axis_index("s")
    wid = sid * NC + cid
    base = wid * BW

    in_copies = [
        pltpu.make_async_copy(u_hbm.at[pl.ds(base, BW)], uidx, s_in),
        pltpu.make_async_copy(i_hbm.at[pl.ds(base, BW)], iidx, s_in),
        pltpu.make_async_copy(c_hbm.at[pl.ds(base, BW)], cidx, s_in),
        pltpu.make_async_copy(hf_hbm.at[pl.ds(base * HIST, BW * HIST)], hidx, s_in),
    ]
    for c in in_copies:
        c.start()
    for c in in_copies:
        c.wait()

    def uic_copies():
        cs = []
        for idx, w, row in ((uidx, wu_hbm, urow), (iidx, wi_hbm, irow),
                            (cidx, wc_hbm, crow)):
            for j in range(NU):
                cs.append(pltpu.make_async_copy(
                    w.at[idx.at[pl.ds(j * UC, UC)]],
                    row.at[pl.ds(j * UC, UC)], s_uic))
        return cs

    for c in uic_copies():
        c.start()

    def group_copies(g, hb, sem):
        return [
            pltpu.make_async_copy(
                wh_hbm.at[hidx.at[pl.ds(g * CH + k * S, S)]],
                hb.at[pl.ds(k * S, S)], sem)
            for k in range(NOPS)
        ]

    for c in group_copies(0, hb0, s0):
        c.start()

    bufs = (hb0, hb1)
    sems = (s0, s1)
    inv = _f32(1.0 / HIST)

    @pl.loop(0, NG, step=2)
    def _groups(gg):
        for b in range(2):
            g = gg + b
            nb = 1 - b

            @pl.when(g + 1 < NG)
            def _():
                for c in group_copies(g + 1, bufs[nb], sems[nb]):
                    c.start()

            for c in group_copies(g, bufs[b], sems[b]):
                c.wait()

            hb = bufs[b]
            for r in range(G):
                o = r * HIST
                a0 = hb[o + 0, :]
                a1 = hb[o + 1, :]
                a2 = hb[o + 2, :]
                a3 = hb[o + 3, :]
                for h in range(4, HIST - 3, 4):
                    a0 = a0 + hb[o + h, :]
                    a1 = a1 + hb[o + h + 1, :]
                    a2 = a2 + hb[o + h + 2, :]
                    a3 = a3 + hb[o + h + 3, :]
                acc = (a0 + a1) + (a2 + a3)
                for h in range(4 * ((HIST - 4) // 4) + 4, HIST):
                    acc = acc + hb[o + h, :]
                hout[g * G + r, :] = acc * inv

    for c in uic_copies():
        c.wait()
    pltpu.sync_copy(urow, ou_hbm.at[pl.ds(base, BW)])
    pltpu.sync_copy(irow, oi_hbm.at[pl.ds(base, BW)])
    pltpu.sync_copy(crow, oc_hbm.at[pl.ds(base, BW)])
    pltpu.sync_copy(hout, oh_hbm.at[pl.ds(base, BW)])


def kernel(user_id, item_id, category, hist_items,
           W_user_id, W_item_id, W_category, W_hist_items):
    hist_flat = hist_items.reshape(-1)
    return _emb_kernel(user_id, item_id, category, hist_flat,
                       W_user_id, W_item_id, W_category, W_hist_items)


# speedup vs baseline: 1.0023x; 1.0023x over previous
import functools

import jax
import jax.numpy as jnp
from jax import lax
from jax.experimental import pallas as pl
from jax.experimental.pallas import tpu as pltpu
from jax.experimental.pallas import tpu_sc as plsc

B = 16384
HIST = 50
D = 16
NC = 2
NS = 16
NW = NC * NS
BW = B // NW
G = 16
CH = G * HIST
S = 800
NOPS = CH // S
NG = BW // G
UC = 512
NU = BW // UC

_mesh = plsc.VectorSubcoreMesh(
    core_axis_name="c", subcore_axis_name="s", num_cores=NC, num_subcores=NS
)

_f32 = jnp.float32
_out_sds = jax.ShapeDtypeStruct((B, D), _f32)


@functools.partial(
    pl.kernel,
    out_type=(_out_sds, _out_sds, _out_sds, _out_sds),
    mesh=_mesh,
    compiler_params=pltpu.CompilerParams(use_tc_tiling_on_sc=False),
    scratch_types=[
        pltpu.VMEM((BW,), jnp.int32),
        pltpu.VMEM((BW,), jnp.int32),
        pltpu.VMEM((BW,), jnp.int32),
        pltpu.VMEM((BW * HIST,), jnp.int32),
        pltpu.VMEM((BW, D), _f32),
        pltpu.VMEM((BW, D), _f32),
        pltpu.VMEM((BW, D), _f32),
        pltpu.VMEM((BW, D), _f32),
        pltpu.VMEM((CH, D), _f32),
        pltpu.VMEM((CH, D), _f32),
        pltpu.SemaphoreType.DMA,
        pltpu.SemaphoreType.DMA,
        pltpu.SemaphoreType.DMA,
        pltpu.SemaphoreType.DMA,
    ],
)
def _emb_kernel(u_hbm, i_hbm, c_hbm, hf_hbm, wu_hbm, wi_hbm, wc_hbm, wh_hbm,
                ou_hbm, oi_hbm, oc_hbm, oh_hbm,
                uidx, iidx, cidx, hidx, urow, irow, crow, hout, hb0, hb1,
                s_in, s_uic, s0, s1):
    cid = lax.axis_index("c")
    sid = lax.axis_index("s")
    wid = sid * NC + cid
    base = wid * BW

    in_copies = [
        pltpu.make_async_copy(u_hbm.at[pl.ds(base, BW)], uidx, s_in),
        pltpu.make_async_copy(i_hbm.at[pl.ds(base, BW)], iidx, s_in),
        pltpu.make_async_copy(c_hbm.at[pl.ds(base, BW)], cidx, s_in),
        pltpu.make_async_copy(hf_hbm.at[pl.ds(base * HIST, BW * HIST)], hidx, s_in),
    ]
    for c in in_copies:
        c.start()
    for c in in_copies:
        c.wait()

    def uic_copies():
        cs = []
        for idx, w, row in ((uidx, wu_hbm, urow), (iidx, wi_hbm, irow),
                            (cidx, wc_hbm, crow)):
            for j in range(NU):
                cs.append(pltpu.make_async_copy(
                    w.at[idx.at[pl.ds(j * UC, UC)]],
                    row.at[pl.ds(j * UC, UC)], s_uic))
        return cs

    for c in uic_copies():
        c.start()

    def group_copies(g, hb, sem):
        return [
            pltpu.make_async_copy(
                wh_hbm.at[hidx.at[pl.ds(g * CH + k * S, S)]],
                hb.at[pl.ds(k * S, S)], sem)
            for k in range(NOPS)
        ]

    for c in group_copies(0, hb0, s0):
        c.start()

    bufs = (hb0, hb1)
    sems = (s0, s1)
    inv = _f32(1.0 / HIST)

    @pl.loop(0, NG, step=2)
    def _groups(gg):
        for b in range(2):
            g = gg + b
            nb = 1 - b

            @pl.when(g + 1 < NG)
            def _():
                for c in group_copies(g + 1, bufs[nb], sems[nb]):
                    c.start()

            for c in group_copies(g, bufs[b], sems[b]):
                c.wait()

            hb = bufs[b]
            for r in range(G):
                o = r * HIST
                a0 = hb[o + 0, :]
                a1 = hb[o + 1, :]
                a2 = hb[o + 2, :]
                a3 = hb[o + 3, :]
                for h in range(4, HIST - 3, 4):
                    a0 = a0 + hb[o + h, :]
                    a1 = a1 + hb[o + h + 1, :]
                    a2 = a2 + hb[o + h + 2, :]
                    a3 = a3 + hb[o + h + 3, :]
                acc = (a0 + a1) + (a2 + a3)
                for h in range(4 * ((HIST - 4) // 4) + 4, HIST):
                    acc = acc + hb[o + h, :]
                hout[g * G + r, :] = acc * inv

    for c in uic_copies():
        c.wait()
    pltpu.sync_copy(urow, ou_hbm.at[pl.ds(base, BW)])
    pltpu.sync_copy(irow, oi_hbm.at[pl.ds(base, BW)])
    pltpu.sync_copy(crow, oc_hbm.at[pl.ds(base, BW)])
    pltpu.sync_copy(hout, oh_hbm.at[pl.ds(base, BW)])


def kernel(user_id, item_id, category, hist_items,
           W_user_id, W_item_id, W_category, W_hist_items):
    hist_flat = hist_items.reshape(-1)
    return _emb_kernel(user_id, item_id, category, hist_flat,
                       W_user_id, W_item_id, W_category, W_hist_items)
